# baseline (speedup 1.0000x reference)
import functools

import jax
import jax.numpy as jnp
from jax import lax
from jax.experimental import pallas as pl
from jax.experimental.pallas import tpu as pltpu
from jax.experimental.pallas import tpu_sc as plsc

VOCAB = 1000000
EMB = 64
ROWS = 16384
COLS = 50
B = ROWS * COLS
NC = 2
NS = 16
NW = NC * NS
B_PER_W = B // NW
CHUNK = 128
STEPS = B_PER_W // CHUNK
ITILES = ROWS // CHUNK

_mesh = plsc.VectorSubcoreMesh(core_axis_name="c", subcore_axis_name="s")


@functools.partial(
    pl.kernel,
    out_type=jax.ShapeDtypeStruct((COLS * 8, ITILES, 8, CHUNK), jnp.float32),
    mesh=_mesh,
    scratch_types=[
        pltpu.VMEM((STEPS, CHUNK), jnp.int32),
        pltpu.VMEM((CHUNK, EMB), jnp.float32),
        pltpu.VMEM((8, 1, 8, CHUNK), jnp.float32),
        pltpu.SemaphoreType.DMA,
    ],
    compiler_params=pltpu.CompilerParams(
        use_tc_tiling_on_sc=False, needs_layout_passes=False
    ),
)
def _gather_kernel(idx_hbm, table_hbm, out_hbm, idx_v, rows_v, tbuf, sem):
    wid = lax.axis_index("s") * NC + lax.axis_index("c")
    pltpu.sync_copy(idx_hbm.at[wid], idx_v)

    lane = lax.iota(jnp.int32, 16)
    row_ids = [lane + 16 * k for k in range(8)]

    @pl.loop(0, STEPS)
    def _(j):
        g = wid * STEPS + j
        c = g // ITILES
        it = g % ITILES
        pltpu.async_copy(table_hbm.at[idx_v.at[j]], rows_v, sem).wait()
        for e in range(EMB):
            col = jnp.full((16,), e, jnp.int32)
            for k in range(8):
                v = plsc.load_gather(rows_v, [row_ids[k], col])
                tbuf[e // 8, 0, e % 8, pl.ds(16 * k, 16)] = v
        pltpu.sync_copy(tbuf, out_hbm.at[pl.ds(c * 8, 8), pl.ds(it, 1)])


def kernel(data, ivectors_weight):
    idx = data.T.reshape(NW, STEPS, CHUNK).astype(jnp.int32)
    out4 = _gather_kernel(idx, ivectors_weight)
    res = (
        out4.reshape(COLS, 8, ITILES, 8, CHUNK)
        .transpose(2, 4, 0, 1, 3)
        .reshape(ROWS, COLS, EMB)
    )
    return res


# speedup vs baseline: 1.6980x; 1.6980x over previous
import functools

import jax
import jax.numpy as jnp
from jax import lax
from jax.experimental import pallas as pl
from jax.experimental.pallas import tpu as pltpu
from jax.experimental.pallas import tpu_sc as plsc

VOCAB = 1000000
EMB = 64
ROWS = 16384
COLS = 50
B = ROWS * COLS
NC = 2
NS = 16
NW = NC * NS
B_PER_W = B // NW
CHUNK = 128
STEPS = B_PER_W // CHUNK
ITILES = ROWS // CHUNK

_mesh = plsc.VectorSubcoreMesh(core_axis_name="c", subcore_axis_name="s")


@functools.partial(
    pl.kernel,
    out_type=jax.ShapeDtypeStruct((COLS * 8, ITILES, 8, CHUNK), jnp.float32),
    mesh=_mesh,
    scratch_types=[
        pltpu.VMEM((STEPS, CHUNK), jnp.int32),
        pltpu.VMEM((CHUNK, EMB), jnp.float32),
        pltpu.VMEM((8, 1, 8, 132), jnp.float32),
        pltpu.SemaphoreType.DMA,
    ],
    compiler_params=pltpu.CompilerParams(
        use_tc_tiling_on_sc=False, needs_layout_passes=False
    ),
)
def _gather_kernel(idx_hbm, table_hbm, out_hbm, idx_v, rows_v, tbuf, sem):
    wid = lax.axis_index("s") * NC + lax.axis_index("c")
    pltpu.sync_copy(idx_hbm.at[wid], idx_v)

    lane = lax.iota(jnp.int32, 16)
    zero16 = jnp.zeros((16,), jnp.int32)
    e_lo = lane % 8
    e_hi = [lane // 8 + 2 * g for g in range(4)]

    @pl.loop(0, STEPS)
    def _(j):
        g = wid * STEPS + j
        c = g // ITILES
        it = g % ITILES
        pltpu.async_copy(table_hbm.at[idx_v.at[j]], rows_v, sem).wait()
        for i in range(CHUNK):
            ivec = jnp.full((16,), i, jnp.int32)
            for gq in range(4):
                v = rows_v[i, pl.ds(16 * gq, 16)]
                plsc.store_scatter(tbuf, [e_hi[gq], zero16, e_lo, ivec], v)
        pltpu.sync_copy(
            tbuf.at[:, :, :, pl.ds(0, CHUNK)],
            out_hbm.at[pl.ds(c * 8, 8), pl.ds(it, 1)],
        )


def kernel(data, ivectors_weight):
    idx = data.T.reshape(NW, STEPS, CHUNK).astype(jnp.int32)
    out4 = _gather_kernel(idx, ivectors_weight)
    res = (
        out4.reshape(COLS, 8, ITILES, 8, CHUNK)
        .transpose(2, 4, 0, 1, 3)
        .reshape(ROWS, COLS, EMB)
    )
    return res


# speedup vs baseline: 1.9017x; 1.1200x over previous
import functools

import jax
import jax.numpy as jnp
from jax import lax
from jax.experimental import pallas as pl
from jax.experimental.pallas import tpu as pltpu
from jax.experimental.pallas import tpu_sc as plsc

VOCAB = 1000000
EMB = 64
ROWS = 16384
COLS = 50
B = ROWS * COLS
NC = 2
NS = 16
NW = NC * NS
B_PER_W = B // NW
CHUNK = 128
STEPS = B_PER_W // CHUNK
ITILES = ROWS // CHUNK
PITCH = 131

_mesh = plsc.VectorSubcoreMesh(core_axis_name="c", subcore_axis_name="s")


@functools.partial(
    pl.kernel,
    out_type=jax.ShapeDtypeStruct((COLS * 8, ITILES, 8, CHUNK), jnp.float32),
    mesh=_mesh,
    scratch_types=[
        pltpu.VMEM((STEPS, CHUNK), jnp.int32),
        pltpu.VMEM((CHUNK, EMB), jnp.float32),
        pltpu.VMEM((CHUNK, EMB), jnp.float32),
        pltpu.VMEM((8, 1, 8, PITCH), jnp.float32),
        pltpu.VMEM((8, 1, 8, PITCH), jnp.float32),
        pltpu.SemaphoreType.DMA,
        pltpu.SemaphoreType.DMA,
        pltpu.SemaphoreType.DMA,
        pltpu.SemaphoreType.DMA,
    ],
    compiler_params=pltpu.CompilerParams(
        use_tc_tiling_on_sc=False, needs_layout_passes=False
    ),
)
def _gather_kernel(idx_hbm, table_hbm, out_hbm, idx_v,
                   rows0, rows1, tb0, tb1, g0, g1, s0, s1):
    rows = (rows0, rows1)
    tbuf = (tb0, tb1)
    gsem = (g0, g1)
    ssem = (s0, s1)

    wid = lax.axis_index("s") * NC + lax.axis_index("c")
    pltpu.sync_copy(idx_hbm.at[wid], idx_v)

    lane = lax.iota(jnp.int32, 16)
    zero16 = jnp.zeros((16,), jnp.int32)
    e_lo = lane % 8
    e_hi = [lane // 8 + 2 * g for g in range(4)]

    def gather(b, j):
        return pltpu.make_async_copy(
            table_hbm.at[idx_v.at[j]], rows[b], gsem[b])

    def store(b, c, it):
        return pltpu.make_async_copy(
            tbuf[b].at[:, :, :, pl.ds(0, CHUNK)],
            out_hbm.at[pl.ds(c * 8, 8), pl.ds(it, 1)],
            ssem[b])

    def transpose(b):
        for i in range(CHUNK):
            ivec = jnp.full((16,), i, jnp.int32)
            for gq in range(4):
                v = rows[b][i, pl.ds(16 * gq, 16)]
                plsc.store_scatter(tbuf[b], [e_hi[gq], zero16, e_lo, ivec], v)

    gather(0, 0).start()
    gather(1, 1).start()

    @pl.loop(0, STEPS, step=2)
    def _(j0):
        for b in range(2):
            j = j0 + b
            g = wid * STEPS + j
            c = g // ITILES
            it = g % ITILES
            gather(b, j).wait()

            @pl.when(j >= 2)
            def _():
                store(b, c, it).wait()

            transpose(b)
            store(b, c, it).start()

            @pl.when(j + 2 < STEPS)
            def _():
                gather(b, j + 2).start()

    store(0, 0, 0).wait()
    store(1, 0, 0).wait()


def kernel(data, ivectors_weight):
    idx = data.T.reshape(NW, STEPS, CHUNK).astype(jnp.int32)
    out4 = _gather_kernel(idx, ivectors_weight)
    res = (
        out4.reshape(COLS, 8, ITILES, 8, CHUNK)
        .transpose(2, 4, 0, 1, 3)
        .reshape(ROWS, COLS, EMB)
    )
    return res


# speedup vs baseline: 2.0294x; 1.0672x over previous
import functools

import jax
import jax.numpy as jnp
from jax import lax
from jax.experimental import pallas as pl
from jax.experimental.pallas import tpu as pltpu
from jax.experimental.pallas import tpu_sc as plsc

VOCAB = 1000000
EMB = 64
ROWS = 16384
COLS = 50
B = ROWS * COLS
NC = 2
NS = 16
NW = NC * NS
B_PER_W = B // NW
CHUNK = 128
STEPS = B_PER_W // CHUNK
ITILES = ROWS // CHUNK
PITCH = 131

_mesh = plsc.VectorSubcoreMesh(core_axis_name="c", subcore_axis_name="s")


@functools.partial(
    pl.kernel,
    out_type=jax.ShapeDtypeStruct((COLS * 8, ITILES, 8, CHUNK), jnp.float32),
    mesh=_mesh,
    scratch_types=[
        pltpu.VMEM((STEPS, CHUNK), jnp.int32),
        pltpu.VMEM((CHUNK, 2 * EMB), jnp.float32),
        pltpu.VMEM((CHUNK, 2 * EMB), jnp.float32),
        pltpu.VMEM((8, 1, 8, PITCH), jnp.float32),
        pltpu.VMEM((8, 1, 8, PITCH), jnp.float32),
        pltpu.SemaphoreType.DMA,
        pltpu.SemaphoreType.DMA,
        pltpu.SemaphoreType.DMA,
        pltpu.SemaphoreType.DMA,
    ],
    compiler_params=pltpu.CompilerParams(
        use_tc_tiling_on_sc=False, needs_layout_passes=False
    ),
)
def _gather_kernel(idx_hbm, table_hbm, out_hbm, idx_v,
                   rows0, rows1, tb0, tb1, g0, g1, s0, s1):
    rows = (rows0, rows1)
    tbuf = (tb0, tb1)
    gsem = (g0, g1)
    ssem = (s0, s1)

    wid = lax.axis_index("s") * NC + lax.axis_index("c")
    pltpu.sync_copy(idx_hbm.at[wid], idx_v)

    lane = lax.iota(jnp.int32, 16)
    zero16 = jnp.zeros((16,), jnp.int32)
    e_lo = lane % 8
    e_hi = [lane // 8 + 2 * g for g in range(4)]

    def gather(b, j):
        return pltpu.make_async_copy(
            table_hbm.at[idx_v.at[j]], rows[b], gsem[b])

    def store(b, c, it):
        return pltpu.make_async_copy(
            tbuf[b].at[:, :, :, pl.ds(0, CHUNK)],
            out_hbm.at[pl.ds(c * 8, 8), pl.ds(it, 1)],
            ssem[b])

    def transpose(b):
        for i in range(CHUNK):
            ivec = jnp.full((16,), i, jnp.int32)
            for gq in range(4):
                v = rows[b][i, pl.ds(16 * gq, 16)]
                plsc.store_scatter(tbuf[b], [e_hi[gq], zero16, e_lo, ivec], v)

    gather(0, 0).start()
    gather(1, 1).start()

    @pl.loop(0, STEPS, step=2)
    def _(j0):
        for b in range(2):
            j = j0 + b
            g = wid * STEPS + j
            c = g // ITILES
            it = g % ITILES
            gather(b, j).wait()

            @pl.when(j >= 2)
            def _():
                store(b, c, it).wait()

            transpose(b)
            store(b, c, it).start()

            @pl.when(j + 2 < STEPS)
            def _():
                gather(b, j + 2).start()

    store(0, 0, 0).wait()
    store(1, 0, 0).wait()


def kernel(data, ivectors_weight):
    idx = data.T.reshape(NW, STEPS, CHUNK).astype(jnp.int32)
    wp = jnp.pad(ivectors_weight, ((0, 0), (0, EMB)))
    out4 = _gather_kernel(idx, wp)
    res = (
        out4.reshape(COLS, 8, ITILES, 8, CHUNK)
        .transpose(2, 4, 0, 1, 3)
        .reshape(ROWS, COLS, EMB)
    )
    return res
